# baseline (speedup 1.0000x reference)
import functools

import jax
import jax.numpy as jnp
from jax import lax
from jax.experimental import pallas as pl
from jax.experimental.pallas import tpu as pltpu
from jax.experimental.pallas import tpu_sc as plsc

N_NODES = 50000
N_REL = 17
D = 64
E = 800000

NC = 2
NS = 16
HALF = N_NODES // NC
PACK_ROWS = N_NODES // 2
ROWS_PER_SUB = 784
ACC_ROWS = NS * ROWS_PER_SUB
JUNK_ROW = ACC_ROWS - 1
CHUNK = 48
CPB = 21
IBLK = CHUNK * CPB
NCH = 1050
EPS = NCH * CHUNK
E_PAD = NS * EPS


def _tanh(x):
    return 2.0 / (1.0 + jnp.exp(-2.0 * x)) - 1.0


def _sc_edge_body(heads, tails, etype, curp, rproj, zeros, out,
                  acc, hblk, tblk, eblk,
                  hrowb, trowb, lidxb, hoffb, toffb, rbb,
                  hraw, traw, rpv, gsem, ssem):
    c = lax.axis_index("c")
    s = lax.axis_index("s")
    base_node = c * HALF
    sbase = s * EPS

    pltpu.sync_copy(rproj, rpv)
    pltpu.sync_copy(zeros, acc.at[pl.ds(s * ROWS_PER_SUB, ROWS_PER_SUB)])
    plsc.subcore_barrier()

    def load_block(blk):
        pltpu.sync_copy(heads.at[pl.ds(sbase + blk * IBLK, IBLK)], hblk)
        pltpu.sync_copy(tails.at[pl.ds(sbase + blk * IBLK, IBLK)], tblk)
        pltpu.sync_copy(etype.at[pl.ds(sbase + blk * IBLK, IBLK)], eblk)

    def stage(ic, slot):
        off = lax.rem(ic, CPB) * CHUNK
        for g in range(CHUNK // 16):
            sl = pl.ds(g * 16, 16)
            dsl = pl.ds(off + g * 16, 16)
            hv = hblk[dsl]
            tv = tblk[dsl]
            ev = eblk[dsl]
            hrowb[slot, sl] = jnp.maximum(hv, 0) >> 1
            trowb[slot, sl] = tv >> 1
            lv = hv - base_node
            ok = (lv >= 0) & (lv < HALF)
            lidxb[slot, sl] = jnp.where(ok, lv >> 1, JUNK_ROW)
            hoffb[slot, sl] = (hv & 1) << 6
            toffb[slot, sl] = (tv & 1) << 6
            rbb[slot, sl] = ev << 6
        pltpu.async_copy(curp.at[hrowb.at[slot]], hraw.at[slot],
                         gsem.at[slot])
        pltpu.async_copy(curp.at[trowb.at[slot]], traw.at[slot],
                         gsem.at[slot])

    iota16 = lax.iota(jnp.int32, 16)

    def _bcast(v, j):
        return v.at[jnp.full((16,), j, jnp.int32)].get(
            mode="promise_in_bounds")

    def _allsum(v):
        for sh in (8, 4, 2, 1):
            v = v + v.at[iota16 ^ sh].get(mode="promise_in_bounds")
        return v

    def chunk_compute(p):
        pltpu.make_async_copy(curp.at[hrowb.at[p]], hraw.at[p],
                              gsem.at[p]).wait()
        pltpu.make_async_copy(curp.at[trowb.at[p]], traw.at[p],
                              gsem.at[p]).wait()
        for g in range(CHUNK // 16):
            sl = pl.ds(g * 16, 16)
            hoff16 = hoffb[p, sl]
            toff16 = toffb[p, sl]
            rb16 = rbb[p, sl]
            for j in range(16):
                e = g * 16 + j
                hsel = _bcast(hoff16, j) != 0
                tsel = _bcast(toff16, j) != 0
                rbv = _bcast(rb16, j)
                sc = None
                for k in range(4):
                    hlo = hraw[p, e, pl.ds(16 * k, 16)]
                    hhi = hraw[p, e, pl.ds(D + 16 * k, 16)]
                    hv = jnp.where(hsel, hhi, hlo)
                    rv = plsc.load_gather(rpv, [rbv + (16 * k + iota16)])
                    v = hv * _tanh(hv + rv)
                    sc = v if sc is None else sc + v
                score = _allsum(sc)
                for k in range(4):
                    tlo = traw[p, e, pl.ds(16 * k, 16)]
                    thi = traw[p, e, pl.ds(D + 16 * k, 16)]
                    w = jnp.where(tsel, thi, tlo) * score
                    zf = jnp.zeros((16,), jnp.float32)
                    traw[p, e, pl.ds(16 * k, 16)] = jnp.where(hsel, zf, w)
                    traw[p, e, pl.ds(D + 16 * k, 16)] = jnp.where(hsel, w, zf)
        pltpu.async_copy(traw.at[p], acc.at[lidxb.at[p]], ssem.at[p],
                         add=True)

    load_block(0)
    stage(0, 0)

    def it(i, carry):
        p = lax.rem(i, 2)
        q = lax.rem(i + 1, 2)

        @pl.when(i + 1 < NCH)
        def _prefetch():
            @pl.when(lax.rem(i + 1, CPB) == 0)
            def _refill():
                load_block(lax.div(i + 1, CPB))

            @pl.when(i >= 1)
            def _drain():
                pltpu.make_async_copy(traw.at[q], acc.at[lidxb.at[q]],
                                      ssem.at[q]).wait()

            stage(i + 1, q)

        chunk_compute(p)
        return carry

    lax.fori_loop(0, NCH, it, 0)
    for k2 in range(2):
        pltpu.make_async_copy(traw.at[k2], acc.at[lidxb.at[k2]],
                              ssem.at[k2]).wait()
    plsc.subcore_barrier()
    pltpu.sync_copy(acc.at[pl.ds(s * ROWS_PER_SUB, ROWS_PER_SUB)],
                    out.at[c, pl.ds(s * ROWS_PER_SUB, ROWS_PER_SUB)])


_sc_edge_kernel = functools.partial(
    pl.kernel,
    out_type=jax.ShapeDtypeStruct((NC, ACC_ROWS, 2 * D), jnp.float32),
    mesh=plsc.VectorSubcoreMesh(core_axis_name="c", subcore_axis_name="s"),
    compiler_params=pltpu.CompilerParams(needs_layout_passes=False),
    scratch_types=[
        pltpu.VMEM_SHARED((ACC_ROWS, 2 * D), jnp.float32),
        pltpu.VMEM((IBLK,), jnp.int32),
        pltpu.VMEM((IBLK,), jnp.int32),
        pltpu.VMEM((IBLK,), jnp.int32),
        pltpu.VMEM((2, CHUNK), jnp.int32),
        pltpu.VMEM((2, CHUNK), jnp.int32),
        pltpu.VMEM((2, CHUNK), jnp.int32),
        pltpu.VMEM((2, CHUNK), jnp.int32),
        pltpu.VMEM((2, CHUNK), jnp.int32),
        pltpu.VMEM((2, CHUNK), jnp.int32),
        pltpu.VMEM((2, CHUNK, 2 * D), jnp.float32),
        pltpu.VMEM((2, CHUNK, 2 * D), jnp.float32),
        pltpu.VMEM((N_REL * D,), jnp.float32),
        pltpu.SemaphoreType.DMA((2,)),
        pltpu.SemaphoreType.DMA((2,)),
    ],
)(_sc_edge_body)


ROWS_BLK = 1000
N_BLKS = N_NODES // ROWS_BLK


def _dense_body(cur_ref, neigh_ref, w1t, b1, w2t, b2, out_ref):
    x = cur_ref[...]
    n = neigh_ref[...]
    se = jnp.dot(x + n, w1t[...], preferred_element_type=jnp.float32) + b1[...]
    pe = jnp.dot(x * n, w2t[...], preferred_element_type=jnp.float32) + b2[...]
    y = se + pe
    y = jnp.where(y > 0, y, 0.01 * y)
    nrm = jnp.sqrt(jnp.sum(y * y, axis=1, keepdims=True))
    out_ref[...] = y / jnp.maximum(nrm, 1e-12)


_dense_kernel = pl.pallas_call(
    _dense_body,
    grid=(N_BLKS,),
    in_specs=[
        pl.BlockSpec((ROWS_BLK, D), lambda b: (b, 0)),
        pl.BlockSpec((ROWS_BLK, D), lambda b: (b, 0)),
        pl.BlockSpec((D, D), lambda b: (0, 0)),
        pl.BlockSpec((1, D), lambda b: (0, 0)),
        pl.BlockSpec((D, D), lambda b: (0, 0)),
        pl.BlockSpec((1, D), lambda b: (0, 0)),
    ],
    out_specs=pl.BlockSpec((ROWS_BLK, D), lambda b: (b, 0)),
    out_shape=jax.ShapeDtypeStruct((N_NODES, D), jnp.float32),
)


def _rproj_body(rel_ref, w0t, b0, w1t, b1, out0, out1):
    r = rel_ref[...]
    out0[...] = jnp.dot(r, w0t[...], preferred_element_type=jnp.float32) + b0[...]
    out1[...] = jnp.dot(r, w1t[...], preferred_element_type=jnp.float32) + b1[...]


_rproj_kernel = pl.pallas_call(
    _rproj_body,
    out_shape=(jax.ShapeDtypeStruct((N_REL, D), jnp.float32),
               jax.ShapeDtypeStruct((N_REL, D), jnp.float32)),
)


def kernel(entity_table, relation_table, rp_w0, rp_b0, rp_w1, rp_b1,
           a1w0, a1b0, a2w0, a2b0, a1w1, a1b1, a2w1, a2b1,
           edge_index, edge_type):
    edge_index = edge_index.astype(jnp.int32)
    pad = E_PAD - E
    heads = jnp.concatenate([edge_index[0],
                             jnp.full((pad,), -1, jnp.int32)])
    tails = jnp.concatenate([edge_index[1],
                             jnp.zeros((pad,), jnp.int32)])
    etype = jnp.concatenate([edge_type.astype(jnp.int32),
                             jnp.zeros((pad,), jnp.int32)])
    zeros = jnp.zeros((ROWS_PER_SUB, 2 * D), jnp.float32)

    rproj0, rproj1 = _rproj_kernel(relation_table, rp_w0.T,
                                   rp_b0.reshape(1, D), rp_w1.T,
                                   rp_b1.reshape(1, D))
    rprojs = (rproj0.reshape(-1), rproj1.reshape(-1))
    a1wts = (a1w0.T, a1w1.T)
    a1bs = (a1b0.reshape(1, D), a1b1.reshape(1, D))
    a2wts = (a2w0.T, a2w1.T)
    a2bs = (a2b0.reshape(1, D), a2b1.reshape(1, D))

    cur = entity_table
    curp = entity_table.reshape(PACK_ROWS, 2 * D)
    outs = [cur]
    for i in range(2):
        neigh_p = _sc_edge_kernel(heads, tails, etype, curp, rprojs[i],
                                  zeros)
        neigh = neigh_p[:, :HALF // 2, :].reshape(N_NODES, D)
        cur = _dense_kernel(cur, neigh, a1wts[i], a1bs[i], a2wts[i], a2bs[i])
        curp = cur.reshape(PACK_ROWS, 2 * D)
        outs.append(cur)
    return jnp.concatenate(outs, axis=1)


# speedup vs baseline: 1.1055x; 1.1055x over previous
import functools

import jax
import jax.numpy as jnp
from jax import lax
from jax.experimental import pallas as pl
from jax.experimental.pallas import tpu as pltpu
from jax.experimental.pallas import tpu_sc as plsc

N_NODES = 50000
N_REL = 17
D = 64
E = 800000

NC = 2
NS = 16
HALF = N_NODES // NC
PACK_ROWS = N_NODES // 2
ROWS_PER_SUB = 784
ACC_ROWS = NS * ROWS_PER_SUB
JUNK_ROW = ACC_ROWS - 1
CHUNK = 48
CPB = 21
IBLK = CHUNK * CPB
NCH = 1050
EPS = NCH * CHUNK
E_PAD = NS * EPS


def _tanh(x):
    return 2.0 / (1.0 + jnp.exp(-2.0 * x)) - 1.0


def _sc_edge_body(heads, tails, etype, curp, rproj, zeros, out,
                  acc, hblk, tblk, eblk,
                  hrowb, trowb, lidxb, hoffb, toffb, rbb,
                  hraw, traw, rpv, gsem, ssem):
    c = lax.axis_index("c")
    s = lax.axis_index("s")
    base_node = c * HALF
    sbase = s * EPS

    pltpu.sync_copy(rproj, rpv)
    pltpu.sync_copy(zeros, acc.at[pl.ds(s * ROWS_PER_SUB, ROWS_PER_SUB)])
    plsc.subcore_barrier()

    def load_block(blk):
        pltpu.sync_copy(heads.at[pl.ds(sbase + blk * IBLK, IBLK)], hblk)
        pltpu.sync_copy(tails.at[pl.ds(sbase + blk * IBLK, IBLK)], tblk)
        pltpu.sync_copy(etype.at[pl.ds(sbase + blk * IBLK, IBLK)], eblk)

    def stage(ic, slot):
        off = lax.rem(ic, CPB) * CHUNK
        for g in range(CHUNK // 16):
            sl = pl.ds(g * 16, 16)
            dsl = pl.ds(off + g * 16, 16)
            hv = hblk[dsl]
            tv = tblk[dsl]
            ev = eblk[dsl]
            hrowb[slot, sl] = jnp.maximum(hv, 0) >> 1
            trowb[slot, sl] = tv >> 1
            lv = hv - base_node
            ok = (lv >= 0) & (lv < HALF)
            lidxb[slot, sl] = jnp.where(ok, lv >> 1, JUNK_ROW)
            hoffb[slot, sl] = (hv & 1) << 6
            toffb[slot, sl] = (tv & 1) << 6
            rbb[slot, sl] = ev << 6
        half = CHUNK // 2
        for lo in (0, half):
            pltpu.async_copy(curp.at[hrowb.at[slot, pl.ds(lo, half)]],
                             hraw.at[slot, pl.ds(lo, half)], gsem.at[slot])
            pltpu.async_copy(curp.at[trowb.at[slot, pl.ds(lo, half)]],
                             traw.at[slot, pl.ds(lo, half)], gsem.at[slot])

    zero16 = jnp.zeros((16,), jnp.float32)

    def chunk_compute(p):
        pltpu.make_async_copy(curp.at[hrowb.at[p]], hraw.at[p],
                              gsem.at[p]).wait()
        pltpu.make_async_copy(curp.at[trowb.at[p]], traw.at[p],
                              gsem.at[p]).wait()
        for g in range(CHUNK // 16):
            sl = pl.ds(g * 16, 16)
            hoff16 = hoffb[p, sl]
            toff16 = toffb[p, sl]
            zoff16 = hoff16 ^ D
            rb16 = rbb[p, sl]
            for j in range(16):
                e = g * 16 + j
                hoff = hoff16[j]
                toff = toff16[j]
                zoff = zoff16[j]
                rb = rb16[j]
                sc = None
                for k in range(4):
                    hv = hraw[p, e, pl.ds(hoff + 16 * k, 16)]
                    rv = rpv[pl.ds(rb + 16 * k, 16)]
                    v = hv * _tanh(hv + rv)
                    sc = v if sc is None else sc + v
                score = jnp.sum(sc)
                ts = [traw[p, e, pl.ds(toff + 16 * k, 16)] for k in range(4)]
                for k in range(4):
                    traw[p, e, pl.ds(hoff + 16 * k, 16)] = ts[k] * score
                    traw[p, e, pl.ds(zoff + 16 * k, 16)] = zero16
        pltpu.async_copy(traw.at[p], acc.at[lidxb.at[p]], ssem.at[p],
                         add=True)

    load_block(0)
    stage(0, 0)

    def it(i, carry):
        p = lax.rem(i, 2)
        q = lax.rem(i + 1, 2)

        @pl.when(i + 1 < NCH)
        def _prefetch():
            @pl.when(lax.rem(i + 1, CPB) == 0)
            def _refill():
                load_block(lax.div(i + 1, CPB))

            @pl.when(i >= 1)
            def _drain():
                pltpu.make_async_copy(traw.at[q], acc.at[lidxb.at[q]],
                                      ssem.at[q]).wait()

            stage(i + 1, q)

        chunk_compute(p)
        return carry

    lax.fori_loop(0, NCH, it, 0)
    for k2 in range(2):
        pltpu.make_async_copy(traw.at[k2], acc.at[lidxb.at[k2]],
                              ssem.at[k2]).wait()
    plsc.subcore_barrier()
    pltpu.sync_copy(acc.at[pl.ds(s * ROWS_PER_SUB, ROWS_PER_SUB)],
                    out.at[c, pl.ds(s * ROWS_PER_SUB, ROWS_PER_SUB)])


_sc_edge_kernel = functools.partial(
    pl.kernel,
    out_type=jax.ShapeDtypeStruct((NC, ACC_ROWS, 2 * D), jnp.float32),
    mesh=plsc.VectorSubcoreMesh(core_axis_name="c", subcore_axis_name="s"),
    compiler_params=pltpu.CompilerParams(needs_layout_passes=False),
    scratch_types=[
        pltpu.VMEM_SHARED((ACC_ROWS, 2 * D), jnp.float32),
        pltpu.VMEM((IBLK,), jnp.int32),
        pltpu.VMEM((IBLK,), jnp.int32),
        pltpu.VMEM((IBLK,), jnp.int32),
        pltpu.VMEM((2, CHUNK), jnp.int32),
        pltpu.VMEM((2, CHUNK), jnp.int32),
        pltpu.VMEM((2, CHUNK), jnp.int32),
        pltpu.VMEM((2, CHUNK), jnp.int32),
        pltpu.VMEM((2, CHUNK), jnp.int32),
        pltpu.VMEM((2, CHUNK), jnp.int32),
        pltpu.VMEM((2, CHUNK, 2 * D), jnp.float32),
        pltpu.VMEM((2, CHUNK, 2 * D), jnp.float32),
        pltpu.VMEM((N_REL * D,), jnp.float32),
        pltpu.SemaphoreType.DMA((2,)),
        pltpu.SemaphoreType.DMA((2,)),
    ],
)(_sc_edge_body)


ROWS_BLK = 1000
N_BLKS = N_NODES // ROWS_BLK


def _dense_body(cur_ref, neigh_ref, w1t, b1, w2t, b2, out_ref):
    x = cur_ref[...]
    n = neigh_ref[...]
    se = jnp.dot(x + n, w1t[...], preferred_element_type=jnp.float32) + b1[...]
    pe = jnp.dot(x * n, w2t[...], preferred_element_type=jnp.float32) + b2[...]
    y = se + pe
    y = jnp.where(y > 0, y, 0.01 * y)
    nrm = jnp.sqrt(jnp.sum(y * y, axis=1, keepdims=True))
    out_ref[...] = y / jnp.maximum(nrm, 1e-12)


_dense_kernel = pl.pallas_call(
    _dense_body,
    grid=(N_BLKS,),
    in_specs=[
        pl.BlockSpec((ROWS_BLK, D), lambda b: (b, 0)),
        pl.BlockSpec((ROWS_BLK, D), lambda b: (b, 0)),
        pl.BlockSpec((D, D), lambda b: (0, 0)),
        pl.BlockSpec((1, D), lambda b: (0, 0)),
        pl.BlockSpec((D, D), lambda b: (0, 0)),
        pl.BlockSpec((1, D), lambda b: (0, 0)),
    ],
    out_specs=pl.BlockSpec((ROWS_BLK, D), lambda b: (b, 0)),
    out_shape=jax.ShapeDtypeStruct((N_NODES, D), jnp.float32),
)


def _rproj_body(rel_ref, w0t, b0, w1t, b1, out0, out1):
    r = rel_ref[...]
    out0[...] = jnp.dot(r, w0t[...], preferred_element_type=jnp.float32) + b0[...]
    out1[...] = jnp.dot(r, w1t[...], preferred_element_type=jnp.float32) + b1[...]


_rproj_kernel = pl.pallas_call(
    _rproj_body,
    out_shape=(jax.ShapeDtypeStruct((N_REL, D), jnp.float32),
               jax.ShapeDtypeStruct((N_REL, D), jnp.float32)),
)


def kernel(entity_table, relation_table, rp_w0, rp_b0, rp_w1, rp_b1,
           a1w0, a1b0, a2w0, a2b0, a1w1, a1b1, a2w1, a2b1,
           edge_index, edge_type):
    edge_index = edge_index.astype(jnp.int32)
    pad = E_PAD - E
    heads = jnp.concatenate([edge_index[0],
                             jnp.full((pad,), -1, jnp.int32)])
    tails = jnp.concatenate([edge_index[1],
                             jnp.zeros((pad,), jnp.int32)])
    etype = jnp.concatenate([edge_type.astype(jnp.int32),
                             jnp.zeros((pad,), jnp.int32)])
    zeros = jnp.zeros((ROWS_PER_SUB, 2 * D), jnp.float32)

    rproj0, rproj1 = _rproj_kernel(relation_table, rp_w0.T,
                                   rp_b0.reshape(1, D), rp_w1.T,
                                   rp_b1.reshape(1, D))
    rprojs = (rproj0.reshape(-1), rproj1.reshape(-1))
    a1wts = (a1w0.T, a1w1.T)
    a1bs = (a1b0.reshape(1, D), a1b1.reshape(1, D))
    a2wts = (a2w0.T, a2w1.T)
    a2bs = (a2b0.reshape(1, D), a2b1.reshape(1, D))

    cur = entity_table
    curp = entity_table.reshape(PACK_ROWS, 2 * D)
    outs = [cur]
    for i in range(2):
        neigh_p = _sc_edge_kernel(heads, tails, etype, curp, rprojs[i],
                                  zeros)
        neigh = neigh_p[:, :HALF // 2, :].reshape(N_NODES, D)
        cur = _dense_kernel(cur, neigh, a1wts[i], a1bs[i], a2wts[i], a2bs[i])
        curp = cur.reshape(PACK_ROWS, 2 * D)
        outs.append(cur)
    return jnp.concatenate(outs, axis=1)


# speedup vs baseline: 1.1152x; 1.0089x over previous
import functools

import jax
import jax.numpy as jnp
from jax import lax
from jax.experimental import pallas as pl
from jax.experimental.pallas import tpu as pltpu
from jax.experimental.pallas import tpu_sc as plsc

N_NODES = 50000
N_REL = 17
D = 64
E = 800000

NC = 2
NS = 16
HALF = N_NODES // NC
PACK_ROWS = N_NODES // 2
ROWS_PER_SUB = 784
ACC_ROWS = NS * ROWS_PER_SUB
JUNK_ROW = ACC_ROWS - 1
CHUNK = 48
CPB = 21
IBLK = CHUNK * CPB
NCH = 1050
EPS = NCH * CHUNK
E_PAD = NS * EPS


def _tanh(x):
    return 2.0 / (1.0 + jnp.exp(-2.0 * x)) - 1.0


def _sc_edge_body(heads, tails, etype, curp, rproj, zeros, out,
                  acc, hblk, tblk, eblk,
                  hrowb, trowb, lidxb, hoffb, toffb, rbb,
                  hraw, traw, rpv, gsem, ssem):
    c = lax.axis_index("c")
    s = lax.axis_index("s")
    base_node = c * HALF
    sbase = s * EPS

    pltpu.sync_copy(rproj, rpv)
    pltpu.sync_copy(zeros, acc.at[pl.ds(s * ROWS_PER_SUB, ROWS_PER_SUB)])
    plsc.subcore_barrier()

    def load_block(blk):
        pltpu.sync_copy(heads.at[pl.ds(sbase + blk * IBLK, IBLK)], hblk)
        pltpu.sync_copy(tails.at[pl.ds(sbase + blk * IBLK, IBLK)], tblk)
        pltpu.sync_copy(etype.at[pl.ds(sbase + blk * IBLK, IBLK)], eblk)

    def stage(ic, slot):
        off = lax.rem(ic, CPB) * CHUNK
        for g in range(CHUNK // 16):
            sl = pl.ds(g * 16, 16)
            dsl = pl.ds(off + g * 16, 16)
            hv = hblk[dsl]
            tv = tblk[dsl]
            ev = eblk[dsl]
            hrowb[slot, sl] = jnp.maximum(hv, 0) >> 1
            trowb[slot, sl] = tv >> 1
            lv = hv - base_node
            ok = (lv >= 0) & (lv < HALF)
            lidxb[slot, sl] = jnp.where(ok, lv >> 1, JUNK_ROW)
            hoffb[slot, sl] = (hv & 1) << 6
            toffb[slot, sl] = (tv & 1) << 6
            rbb[slot, sl] = ev << 6
        pltpu.async_copy(curp.at[hrowb.at[slot]], hraw.at[slot],
                         gsem.at[slot])
        pltpu.async_copy(curp.at[trowb.at[slot]], traw.at[slot],
                         gsem.at[slot])

    zero16 = jnp.zeros((16,), jnp.float32)

    def chunk_compute(p):
        pltpu.make_async_copy(curp.at[hrowb.at[p]], hraw.at[p],
                              gsem.at[p]).wait()
        pltpu.make_async_copy(curp.at[trowb.at[p]], traw.at[p],
                              gsem.at[p]).wait()
        for g in range(CHUNK // 16):
            sl = pl.ds(g * 16, 16)
            hoff16 = hoffb[p, sl]
            toff16 = toffb[p, sl]
            zoff16 = hoff16 ^ D
            rb16 = rbb[p, sl]
            for j in range(16):
                e = g * 16 + j
                hoff = hoff16[j]
                toff = toff16[j]
                zoff = zoff16[j]
                rb = rb16[j]
                sc = None
                for k in range(4):
                    hv = hraw[p, e, pl.ds(hoff + 16 * k, 16)]
                    rv = rpv[pl.ds(rb + 16 * k, 16)]
                    v = hv * _tanh(hv + rv)
                    sc = v if sc is None else sc + v
                score = jnp.sum(sc)
                ts = [traw[p, e, pl.ds(toff + 16 * k, 16)] for k in range(4)]
                for k in range(4):
                    traw[p, e, pl.ds(hoff + 16 * k, 16)] = ts[k] * score
                    traw[p, e, pl.ds(zoff + 16 * k, 16)] = zero16
        pltpu.async_copy(traw.at[p], acc.at[lidxb.at[p]], ssem.at[p],
                         add=True)

    load_block(0)
    stage(0, 0)

    def it(i, carry):
        p = lax.rem(i, 2)
        q = lax.rem(i + 1, 2)

        @pl.when(i + 1 < NCH)
        def _prefetch():
            @pl.when(lax.rem(i + 1, CPB) == 0)
            def _refill():
                load_block(lax.div(i + 1, CPB))

            @pl.when(i >= 1)
            def _drain():
                pltpu.make_async_copy(traw.at[q], acc.at[lidxb.at[q]],
                                      ssem.at[q]).wait()

            stage(i + 1, q)

        chunk_compute(p)
        return carry

    lax.fori_loop(0, NCH, it, 0)
    for k2 in range(2):
        pltpu.make_async_copy(traw.at[k2], acc.at[lidxb.at[k2]],
                              ssem.at[k2]).wait()
    plsc.subcore_barrier()
    pltpu.sync_copy(acc.at[pl.ds(s * ROWS_PER_SUB, ROWS_PER_SUB)],
                    out.at[c, pl.ds(s * ROWS_PER_SUB, ROWS_PER_SUB)])


_sc_edge_kernel = functools.partial(
    pl.kernel,
    out_type=jax.ShapeDtypeStruct((NC, ACC_ROWS, 2 * D), jnp.float32),
    mesh=plsc.VectorSubcoreMesh(core_axis_name="c", subcore_axis_name="s"),
    compiler_params=pltpu.CompilerParams(needs_layout_passes=False),
    scratch_types=[
        pltpu.VMEM_SHARED((ACC_ROWS, 2 * D), jnp.float32),
        pltpu.VMEM((IBLK,), jnp.int32),
        pltpu.VMEM((IBLK,), jnp.int32),
        pltpu.VMEM((IBLK,), jnp.int32),
        pltpu.VMEM((2, CHUNK), jnp.int32),
        pltpu.VMEM((2, CHUNK), jnp.int32),
        pltpu.VMEM((2, CHUNK), jnp.int32),
        pltpu.VMEM((2, CHUNK), jnp.int32),
        pltpu.VMEM((2, CHUNK), jnp.int32),
        pltpu.VMEM((2, CHUNK), jnp.int32),
        pltpu.VMEM((2, CHUNK, 2 * D), jnp.float32),
        pltpu.VMEM((2, CHUNK, 2 * D), jnp.float32),
        pltpu.VMEM((N_REL * D,), jnp.float32),
        pltpu.SemaphoreType.DMA((2,)),
        pltpu.SemaphoreType.DMA((2,)),
    ],
)(_sc_edge_body)


ROWS_BLK = 1000
N_BLKS = N_NODES // ROWS_BLK


def _dense_body(cur_ref, neigh_ref, w1t, b1, w2t, b2, out_ref):
    x = cur_ref[...]
    n = neigh_ref[...]
    se = jnp.dot(x + n, w1t[...], preferred_element_type=jnp.float32) + b1[...]
    pe = jnp.dot(x * n, w2t[...], preferred_element_type=jnp.float32) + b2[...]
    y = se + pe
    y = jnp.where(y > 0, y, 0.01 * y)
    nrm = jnp.sqrt(jnp.sum(y * y, axis=1, keepdims=True))
    out_ref[...] = y / jnp.maximum(nrm, 1e-12)


_dense_kernel = pl.pallas_call(
    _dense_body,
    grid=(N_BLKS,),
    in_specs=[
        pl.BlockSpec((ROWS_BLK, D), lambda b: (b, 0)),
        pl.BlockSpec((ROWS_BLK, D), lambda b: (b, 0)),
        pl.BlockSpec((D, D), lambda b: (0, 0)),
        pl.BlockSpec((1, D), lambda b: (0, 0)),
        pl.BlockSpec((D, D), lambda b: (0, 0)),
        pl.BlockSpec((1, D), lambda b: (0, 0)),
    ],
    out_specs=pl.BlockSpec((ROWS_BLK, D), lambda b: (b, 0)),
    out_shape=jax.ShapeDtypeStruct((N_NODES, D), jnp.float32),
)


def _rproj_body(rel_ref, w0t, b0, w1t, b1, out0, out1):
    r = rel_ref[...]
    out0[...] = jnp.dot(r, w0t[...], preferred_element_type=jnp.float32) + b0[...]
    out1[...] = jnp.dot(r, w1t[...], preferred_element_type=jnp.float32) + b1[...]


_rproj_kernel = pl.pallas_call(
    _rproj_body,
    out_shape=(jax.ShapeDtypeStruct((N_REL, D), jnp.float32),
               jax.ShapeDtypeStruct((N_REL, D), jnp.float32)),
)


def kernel(entity_table, relation_table, rp_w0, rp_b0, rp_w1, rp_b1,
           a1w0, a1b0, a2w0, a2b0, a1w1, a1b1, a2w1, a2b1,
           edge_index, edge_type):
    edge_index = edge_index.astype(jnp.int32)
    pad = E_PAD - E
    heads = jnp.concatenate([edge_index[0],
                             jnp.full((pad,), -1, jnp.int32)])
    tails = jnp.concatenate([edge_index[1],
                             jnp.zeros((pad,), jnp.int32)])
    etype = jnp.concatenate([edge_type.astype(jnp.int32),
                             jnp.zeros((pad,), jnp.int32)])
    zeros = jnp.zeros((ROWS_PER_SUB, 2 * D), jnp.float32)

    rproj0, rproj1 = _rproj_kernel(relation_table, rp_w0.T,
                                   rp_b0.reshape(1, D), rp_w1.T,
                                   rp_b1.reshape(1, D))
    rprojs = (rproj0.reshape(-1), rproj1.reshape(-1))
    a1wts = (a1w0.T, a1w1.T)
    a1bs = (a1b0.reshape(1, D), a1b1.reshape(1, D))
    a2wts = (a2w0.T, a2w1.T)
    a2bs = (a2b0.reshape(1, D), a2b1.reshape(1, D))

    cur = entity_table
    curp = entity_table.reshape(PACK_ROWS, 2 * D)
    outs = [cur]
    for i in range(2):
        neigh_p = _sc_edge_kernel(heads, tails, etype, curp, rprojs[i],
                                  zeros)
        neigh = neigh_p[:, :HALF // 2, :].reshape(N_NODES, D)
        cur = _dense_kernel(cur, neigh, a1wts[i], a1bs[i], a2wts[i], a2bs[i])
        curp = cur.reshape(PACK_ROWS, 2 * D)
        outs.append(cur)
    return jnp.concatenate(outs, axis=1)


# speedup vs baseline: 2.0167x; 1.8083x over previous
import functools

import jax
import jax.numpy as jnp
from jax import lax
from jax.experimental import pallas as pl
from jax.experimental.pallas import tpu as pltpu
from jax.experimental.pallas import tpu_sc as plsc

N_NODES = 50000
N_REL = 17
D = 64
E = 800000

NC = 2
NS = 16
HALF = N_NODES // NC
PACK_ROWS = N_NODES // 2
ROWS_PER_SUB = 784
ACC_ROWS = NS * ROWS_PER_SUB
JUNK_ROW = ACC_ROWS - 1
CHUNK = 48
CPB = 21
IBLK = CHUNK * CPB
NCH = 1050
EPS = NCH * CHUNK
E_PAD = NS * EPS


def _tanh(x):
    return 2.0 / (1.0 + jnp.exp(-2.0 * x)) - 1.0


ROWMASK = (1 << 20) - 1


def _sc_edge_body(heads, tails, etype, curp, rproj, zeros, out,
                  acc, hblk, tblk, eblk,
                  hrowb, trowb, lidxb,
                  hraw, traw, rpv, gsem, ssem):
    c = lax.axis_index("c")
    s = lax.axis_index("s")
    base_node = c * HALF
    base2 = c * (HALF // 2)
    sbase = s * EPS
    iota16 = lax.iota(jnp.int32, 16)

    pltpu.sync_copy(rproj, rpv)
    pltpu.sync_copy(zeros, acc.at[pl.ds(s * ROWS_PER_SUB, ROWS_PER_SUB)])
    plsc.subcore_barrier()

    def load_and_compact(blk):
        pltpu.sync_copy(heads.at[pl.ds(sbase + blk * IBLK, IBLK)], hblk)
        pltpu.sync_copy(tails.at[pl.ds(sbase + blk * IBLK, IBLK)], tblk)
        pltpu.sync_copy(etype.at[pl.ds(sbase + blk * IBLK, IBLK)], eblk)

        def cstep(g, wp):
            sl = pl.ds(g * 16, 16)
            hv = hblk[sl]
            tv = tblk[sl]
            ev = eblk[sl]
            lv = hv - base_node
            ok = (lv >= 0) & (lv < HALF)
            dst = pl.ds(wp, 16)
            plsc.store_compressed(hblk.at[dst],
                                  (jnp.maximum(hv, 0) >> 1) |
                                  ((hv & 1) << 20), mask=ok)
            plsc.store_compressed(tblk.at[dst],
                                  (tv >> 1) | ((tv & 1) << 20), mask=ok)
            plsc.store_compressed(eblk.at[dst], ev << 6, mask=ok)
            return wp + plsc.all_reduce_population_count(ok)[0]

        return lax.fori_loop(0, IBLK // 16, cstep, 0)

    def stage(ic, slot, wp):
        off = ic * CHUNK
        for g in range(CHUNK // 16):
            sl = pl.ds(g * 16, 16)
            dsl = pl.ds(off + g * 16, 16)
            av = hblk[dsl]
            bv = tblk[dsl]
            pad = (off + g * 16 + iota16) >= wp
            ar = av & ROWMASK
            hrowb[slot, sl] = jnp.minimum(ar, PACK_ROWS - 1)
            trowb[slot, sl] = jnp.minimum(bv & ROWMASK, PACK_ROWS - 1)
            lidxb[slot, sl] = jnp.where(pad, JUNK_ROW, ar - base2)
        pltpu.async_copy(curp.at[hrowb.at[slot]], hraw.at[slot],
                         gsem.at[slot])
        pltpu.async_copy(curp.at[trowb.at[slot]], traw.at[slot],
                         gsem.at[slot])

    zero16 = jnp.zeros((16,), jnp.float32)

    def chunk_compute(ic, p):
        pltpu.make_async_copy(curp.at[hrowb.at[p]], hraw.at[p],
                              gsem.at[p]).wait()
        pltpu.make_async_copy(curp.at[trowb.at[p]], traw.at[p],
                              gsem.at[p]).wait()
        off = ic * CHUNK
        for g in range(CHUNK // 16):
            dsl = pl.ds(off + g * 16, 16)
            hoff16 = ((hblk[dsl] >> 20) & 1) << 6
            toff16 = ((tblk[dsl] >> 20) & 1) << 6
            zoff16 = hoff16 ^ D
            rb16 = jnp.minimum(eblk[dsl], (N_REL - 1) * D)
            for j in range(16):
                e = g * 16 + j
                hoff = hoff16[j]
                toff = toff16[j]
                zoff = zoff16[j]
                rb = rb16[j]
                sc = None
                for k in range(4):
                    hv = hraw[p, e, pl.ds(hoff + 16 * k, 16)]
                    rv = rpv[pl.ds(rb + 16 * k, 16)]
                    v = hv * _tanh(hv + rv)
                    sc = v if sc is None else sc + v
                score = jnp.sum(sc)
                ts = [traw[p, e, pl.ds(toff + 16 * k, 16)] for k in range(4)]
                for k in range(4):
                    traw[p, e, pl.ds(hoff + 16 * k, 16)] = ts[k] * score
                    traw[p, e, pl.ds(zoff + 16 * k, 16)] = zero16
        pltpu.async_copy(traw.at[p], acc.at[lidxb.at[p]], ssem.at[p],
                         add=True)

    def block_step(b, gc):
        wp = load_and_compact(b)
        nch = lax.div(wp + (CHUNK - 1), CHUNK)

        @pl.when(nch > 0)
        def _prologue():
            @pl.when(gc >= 2)
            def _drain0():
                pltpu.make_async_copy(traw.at[lax.rem(gc, 2)],
                                      acc.at[lidxb.at[lax.rem(gc, 2)]],
                                      ssem.at[lax.rem(gc, 2)]).wait()

            stage(0, lax.rem(gc, 2), wp)

        def it(j, carry):
            p = lax.rem(gc + j, 2)
            q = lax.rem(gc + j + 1, 2)

            @pl.when(j + 1 < nch)
            def _prefetch():
                @pl.when(gc + j >= 1)
                def _drain():
                    pltpu.make_async_copy(traw.at[q], acc.at[lidxb.at[q]],
                                          ssem.at[q]).wait()

                stage(j + 1, q, wp)

            chunk_compute(j, p)
            return carry

        lax.fori_loop(0, nch, it, 0)
        return gc + nch

    lax.fori_loop(0, NCH // CPB, block_step, 0)
    for k2 in range(2):
        pltpu.make_async_copy(traw.at[k2], acc.at[lidxb.at[k2]],
                              ssem.at[k2]).wait()
    plsc.subcore_barrier()
    pltpu.sync_copy(acc.at[pl.ds(s * ROWS_PER_SUB, ROWS_PER_SUB)],
                    out.at[c, pl.ds(s * ROWS_PER_SUB, ROWS_PER_SUB)])


_sc_edge_kernel = functools.partial(
    pl.kernel,
    out_type=jax.ShapeDtypeStruct((NC, ACC_ROWS, 2 * D), jnp.float32),
    mesh=plsc.VectorSubcoreMesh(core_axis_name="c", subcore_axis_name="s"),
    compiler_params=pltpu.CompilerParams(needs_layout_passes=False),
    scratch_types=[
        pltpu.VMEM_SHARED((ACC_ROWS, 2 * D), jnp.float32),
        pltpu.VMEM((IBLK,), jnp.int32),
        pltpu.VMEM((IBLK,), jnp.int32),
        pltpu.VMEM((IBLK,), jnp.int32),
        pltpu.VMEM((2, CHUNK), jnp.int32),
        pltpu.VMEM((2, CHUNK), jnp.int32),
        pltpu.VMEM((2, CHUNK), jnp.int32),
        pltpu.VMEM((2, CHUNK, 2 * D), jnp.float32),
        pltpu.VMEM((2, CHUNK, 2 * D), jnp.float32),
        pltpu.VMEM((N_REL * D,), jnp.float32),
        pltpu.SemaphoreType.DMA((2,)),
        pltpu.SemaphoreType.DMA((2,)),
    ],
)(_sc_edge_body)


ROWS_BLK = 1000
N_BLKS = N_NODES // ROWS_BLK


def _dense_body(cur_ref, neigh_ref, w1t, b1, w2t, b2, out_ref):
    x = cur_ref[...]
    n = neigh_ref[...]
    se = jnp.dot(x + n, w1t[...], preferred_element_type=jnp.float32) + b1[...]
    pe = jnp.dot(x * n, w2t[...], preferred_element_type=jnp.float32) + b2[...]
    y = se + pe
    y = jnp.where(y > 0, y, 0.01 * y)
    nrm = jnp.sqrt(jnp.sum(y * y, axis=1, keepdims=True))
    out_ref[...] = y / jnp.maximum(nrm, 1e-12)


_dense_kernel = pl.pallas_call(
    _dense_body,
    grid=(N_BLKS,),
    in_specs=[
        pl.BlockSpec((ROWS_BLK, D), lambda b: (b, 0)),
        pl.BlockSpec((ROWS_BLK, D), lambda b: (b, 0)),
        pl.BlockSpec((D, D), lambda b: (0, 0)),
        pl.BlockSpec((1, D), lambda b: (0, 0)),
        pl.BlockSpec((D, D), lambda b: (0, 0)),
        pl.BlockSpec((1, D), lambda b: (0, 0)),
    ],
    out_specs=pl.BlockSpec((ROWS_BLK, D), lambda b: (b, 0)),
    out_shape=jax.ShapeDtypeStruct((N_NODES, D), jnp.float32),
)


def _rproj_body(rel_ref, w0t, b0, w1t, b1, out0, out1):
    r = rel_ref[...]
    out0[...] = jnp.dot(r, w0t[...], preferred_element_type=jnp.float32) + b0[...]
    out1[...] = jnp.dot(r, w1t[...], preferred_element_type=jnp.float32) + b1[...]


_rproj_kernel = pl.pallas_call(
    _rproj_body,
    out_shape=(jax.ShapeDtypeStruct((N_REL, D), jnp.float32),
               jax.ShapeDtypeStruct((N_REL, D), jnp.float32)),
)


def kernel(entity_table, relation_table, rp_w0, rp_b0, rp_w1, rp_b1,
           a1w0, a1b0, a2w0, a2b0, a1w1, a1b1, a2w1, a2b1,
           edge_index, edge_type):
    edge_index = edge_index.astype(jnp.int32)
    pad = E_PAD - E
    heads = jnp.concatenate([edge_index[0],
                             jnp.full((pad,), -1, jnp.int32)])
    tails = jnp.concatenate([edge_index[1],
                             jnp.zeros((pad,), jnp.int32)])
    etype = jnp.concatenate([edge_type.astype(jnp.int32),
                             jnp.zeros((pad,), jnp.int32)])
    zeros = jnp.zeros((ROWS_PER_SUB, 2 * D), jnp.float32)

    rproj0, rproj1 = _rproj_kernel(relation_table, rp_w0.T,
                                   rp_b0.reshape(1, D), rp_w1.T,
                                   rp_b1.reshape(1, D))
    rprojs = (rproj0.reshape(-1), rproj1.reshape(-1))
    a1wts = (a1w0.T, a1w1.T)
    a1bs = (a1b0.reshape(1, D), a1b1.reshape(1, D))
    a2wts = (a2w0.T, a2w1.T)
    a2bs = (a2b0.reshape(1, D), a2b1.reshape(1, D))

    cur = entity_table
    curp = entity_table.reshape(PACK_ROWS, 2 * D)
    outs = [cur]
    for i in range(2):
        neigh_p = _sc_edge_kernel(heads, tails, etype, curp, rprojs[i],
                                  zeros)
        neigh = neigh_p[:, :HALF // 2, :].reshape(N_NODES, D)
        cur = _dense_kernel(cur, neigh, a1wts[i], a1bs[i], a2wts[i], a2bs[i])
        curp = cur.reshape(PACK_ROWS, 2 * D)
        outs.append(cur)
    return jnp.concatenate(outs, axis=1)
